# baseline (device time: 59587 ns/iter reference)
import jax
import jax.numpy as jnp
from jax import lax
from jax.experimental import pallas as pl
from jax.experimental.pallas import tpu as pltpu

N_DEV = 4
B_LOC = 2
SQ = 256
SKV = 256
HQ = 16
DH = 64
D_MODEL = 512
D_QK = HQ * DH
H_CHUNK = D_QK // N_DEV


def kernel(x, Wq, K_ext, V_ext, Wo):
    pos = lax.axis_index("i")
    K_loc = lax.dynamic_slice_in_dim(K_ext, pos * B_LOC, B_LOC, axis=0)
    V_loc = lax.dynamic_slice_in_dim(V_ext, pos * B_LOC, B_LOC, axis=0)
    Kt = K_loc.transpose(0, 2, 1, 3).reshape(B_LOC * HQ, SKV, DH)
    Vt = V_loc.transpose(0, 2, 1, 3).reshape(B_LOC * HQ, SKV, DH)

    def body(x_ref, wq_ref, k_ref, v_ref, wo_ref, out_ref,
             wq_comm, wo_comm, wq_full, wo_full, ctx_scr,
             wq_ss, wq_rs, wo_ss, wo_rs):
        my = lax.axis_index("i")
        left = lax.rem(my + (N_DEV - 1), N_DEV)
        right = lax.rem(my + 1, N_DEV)

        barrier = pltpu.get_barrier_semaphore()
        for nbr in (left, right):
            pl.semaphore_signal(
                barrier, inc=1,
                device_id=(nbr,), device_id_type=pl.DeviceIdType.MESH,
            )
        pl.semaphore_wait(barrier, 2)

        wq_comm[0] = wq_ref[...]
        wo_comm[0] = wo_ref[...]
        wq_full[:, pl.ds(my * H_CHUNK, H_CHUNK)] = wq_ref[...]
        wo_full[pl.ds(my * H_CHUNK, H_CHUNK), :] = wo_ref[...]

        for h in range(N_DEV - 1):
            rq = pltpu.make_async_remote_copy(
                src_ref=wq_comm.at[h], dst_ref=wq_comm.at[h + 1],
                send_sem=wq_ss.at[h], recv_sem=wq_rs.at[h],
                device_id=(right,), device_id_type=pl.DeviceIdType.MESH,
            )
            ro = pltpu.make_async_remote_copy(
                src_ref=wo_comm.at[h], dst_ref=wo_comm.at[h + 1],
                send_sem=wo_ss.at[h], recv_sem=wo_rs.at[h],
                device_id=(right,), device_id_type=pl.DeviceIdType.MESH,
            )
            rq.start()
            ro.start()
            rq.wait()
            ro.wait()
            origin = lax.rem(my + (N_DEV - 1) - h, N_DEV)
            wq_full[:, pl.ds(origin * H_CHUNK, H_CHUNK)] = wq_comm[h + 1]
            wo_full[pl.ds(origin * H_CHUNK, H_CHUNK), :] = wo_comm[h + 1]

        row = lax.broadcasted_iota(jnp.int32, (SQ, SKV), 0) // 64
        col = lax.broadcasted_iota(jnp.int32, (SQ, SKV), 1) // 64
        mask = (row == col) | (col == 0) | (lax.rem(row + col, 3) == 0)
        neg = jnp.float32(-1e9)

        def mm(a, b):
            return lax.dot_general(
                a, b, dimension_numbers=(((1,), (0,)), ((), ())),
                preferred_element_type=jnp.float32,
            )

        def mm_t(a, b):
            return lax.dot_general(
                a, b, dimension_numbers=(((1,), (1,)), ((), ())),
                preferred_element_type=jnp.float32,
            )

        for b in range(B_LOC):
            q_b = mm(x_ref[b], wq_full[...])
            for hh in range(HQ):
                q = q_b[:, hh * DH:(hh + 1) * DH]
                k = k_ref[b * HQ + hh]
                s = mm_t(q, k) * jnp.float32(0.125)
                s = jnp.where(mask, s, neg)
                m = jnp.max(s, axis=1, keepdims=True)
                w = jnp.exp(s - m)
                w = w / jnp.sum(w, axis=1, keepdims=True)
                ctx_scr[:, hh * DH:(hh + 1) * DH] = mm(w, v_ref[b * HQ + hh])
            out_ref[b, :, :] = mm(ctx_scr[...], wo_full[...])

    return pl.pallas_call(
        body,
        out_shape=jax.ShapeDtypeStruct((B_LOC, SQ, D_MODEL), jnp.float32),
        in_specs=[pl.BlockSpec(memory_space=pltpu.VMEM)] * 5,
        out_specs=pl.BlockSpec(memory_space=pltpu.VMEM),
        scratch_shapes=[
            pltpu.VMEM((N_DEV, D_MODEL, H_CHUNK), jnp.float32),
            pltpu.VMEM((N_DEV, H_CHUNK, D_MODEL), jnp.float32),
            pltpu.VMEM((D_MODEL, D_QK), jnp.float32),
            pltpu.VMEM((D_QK, D_MODEL), jnp.float32),
            pltpu.VMEM((SQ, D_QK), jnp.float32),
            pltpu.SemaphoreType.DMA((N_DEV - 1,)),
            pltpu.SemaphoreType.DMA((N_DEV - 1,)),
            pltpu.SemaphoreType.DMA((N_DEV - 1,)),
            pltpu.SemaphoreType.DMA((N_DEV - 1,)),
        ],
        compiler_params=pltpu.CompilerParams(collective_id=0),
    )(x, Wq, Kt, Vt, Wo)


# device time: 27130 ns/iter; 2.1964x vs baseline; 2.1964x over previous
import jax
import jax.numpy as jnp
from jax import lax
from jax.experimental import pallas as pl
from jax.experimental.pallas import tpu as pltpu

N_DEV = 4
B_LOC = 2
SQ = 256
SKV = 256
HQ = 16
H_LOC = HQ // N_DEV
DH = 64
D_MODEL = 512
D_QK = HQ * DH
H_CHUNK = D_QK // N_DEV
BF = jnp.bfloat16


def kernel(x, Wq, K_ext, V_ext, Wo):
    pos = lax.axis_index("i")
    K_loc = lax.dynamic_slice_in_dim(K_ext, pos * B_LOC, B_LOC, axis=0)
    V_loc = lax.dynamic_slice_in_dim(V_ext, pos * B_LOC, B_LOC, axis=0)

    def arrange(a):
        a = a.transpose(0, 2, 1, 3).reshape(B_LOC, N_DEV, H_LOC, SKV, DH)
        return a.transpose(1, 0, 2, 3, 4).astype(BF)

    Kt = arrange(K_loc)
    Vt = arrange(V_loc)
    xb = x.astype(BF)
    wq_b = Wq.astype(BF)
    wo_b = Wo.astype(BF)

    def body(x_ref, wq_ref, k_ref, v_ref, wo_ref, out_ref,
             wq_g, wo_g, ctx_scr, wq_ss, wq_rs, wo_ss, wo_rs):
        my = lax.axis_index("i")
        left = lax.rem(my + (N_DEV - 1), N_DEV)
        right = lax.rem(my + 1, N_DEV)
        opp = lax.rem(my + 2, N_DEV)

        barrier = pltpu.get_barrier_semaphore()
        for nbr in (left, right):
            pl.semaphore_signal(
                barrier, inc=1,
                device_id=(nbr,), device_id_type=pl.DeviceIdType.MESH,
            )
        pl.semaphore_wait(barrier, 2)

        row = lax.broadcasted_iota(jnp.int32, (SQ, SKV), 0) // 64
        col = lax.broadcasted_iota(jnp.int32, (SQ, SKV), 1) // 64
        mask = (row == col) | (col == 0) | (lax.rem(row + col, 3) == 0)
        neg = jnp.float32(-1e9)

        def mm(a, b):
            return lax.dot_general(
                a, b, dimension_numbers=(((1,), (0,)), ((), ())),
                preferred_element_type=jnp.float32,
            )

        def mm_t(a, b):
            return lax.dot_general(
                a, b, dimension_numbers=(((1,), (1,)), ((), ())),
                preferred_element_type=jnp.float32,
            )

        def do_chunk(p, wq_c, wo_c, first):
            for b in range(B_LOC):
                qp = mm(x_ref[b], wq_c)
                for hh in range(H_LOC):
                    q = qp[:, hh * DH:(hh + 1) * DH].astype(BF)
                    k = k_ref[p, b, hh]
                    s = mm_t(q, k) * jnp.float32(0.125)
                    s = jnp.where(mask, s, neg)
                    m = jnp.max(s, axis=1, keepdims=True)
                    w = jnp.exp(s - m)
                    w = (w / jnp.sum(w, axis=1, keepdims=True)).astype(BF)
                    ctx_scr[:, hh * DH:(hh + 1) * DH] = mm(
                        w, v_ref[p, b, hh]).astype(BF)
                partial = mm(ctx_scr[...], wo_c)
                if first:
                    out_ref[b, :, :] = partial
                else:
                    out_ref[b, :, :] = out_ref[b, :, :] + partial

        hop0 = []
        for d, tgt in ((0, right), (1, left)):
            rq = pltpu.make_async_remote_copy(
                src_ref=wq_ref, dst_ref=wq_g.at[my],
                send_sem=wq_ss.at[0, d], recv_sem=wq_rs.at[0, d],
                device_id=(tgt,), device_id_type=pl.DeviceIdType.MESH,
            )
            ro = pltpu.make_async_remote_copy(
                src_ref=wo_ref, dst_ref=wo_g.at[my],
                send_sem=wo_ss.at[0, d], recv_sem=wo_rs.at[0, d],
                device_id=(tgt,), device_id_type=pl.DeviceIdType.MESH,
            )
            rq.start()
            ro.start()
            hop0 += [rq, ro]

        do_chunk(my, wq_ref[...], wo_ref[...], first=True)

        for r in hop0:
            r.wait()

        hop1 = []
        rq = pltpu.make_async_remote_copy(
            src_ref=wq_g.at[left, pl.ds(0, D_MODEL // 2)],
            dst_ref=wq_g.at[left, pl.ds(0, D_MODEL // 2)],
            send_sem=wq_ss.at[1, 0], recv_sem=wq_rs.at[1, 0],
            device_id=(right,), device_id_type=pl.DeviceIdType.MESH,
        )
        ro = pltpu.make_async_remote_copy(
            src_ref=wo_g.at[left, pl.ds(0, H_CHUNK // 2)],
            dst_ref=wo_g.at[left, pl.ds(0, H_CHUNK // 2)],
            send_sem=wo_ss.at[1, 0], recv_sem=wo_rs.at[1, 0],
            device_id=(right,), device_id_type=pl.DeviceIdType.MESH,
        )
        rq.start()
        ro.start()
        hop1 += [rq, ro]
        rq = pltpu.make_async_remote_copy(
            src_ref=wq_g.at[right, pl.ds(D_MODEL // 2, D_MODEL // 2)],
            dst_ref=wq_g.at[right, pl.ds(D_MODEL // 2, D_MODEL // 2)],
            send_sem=wq_ss.at[1, 1], recv_sem=wq_rs.at[1, 1],
            device_id=(left,), device_id_type=pl.DeviceIdType.MESH,
        )
        ro = pltpu.make_async_remote_copy(
            src_ref=wo_g.at[right, pl.ds(H_CHUNK // 2, H_CHUNK // 2)],
            dst_ref=wo_g.at[right, pl.ds(H_CHUNK // 2, H_CHUNK // 2)],
            send_sem=wo_ss.at[1, 1], recv_sem=wo_rs.at[1, 1],
            device_id=(left,), device_id_type=pl.DeviceIdType.MESH,
        )
        rq.start()
        ro.start()
        hop1 += [rq, ro]

        do_chunk(left, wq_g[left], wo_g[left], first=False)
        do_chunk(right, wq_g[right], wo_g[right], first=False)

        for r in hop1:
            r.wait()

        do_chunk(opp, wq_g[opp], wo_g[opp], first=False)

    return pl.pallas_call(
        body,
        out_shape=jax.ShapeDtypeStruct((B_LOC, SQ, D_MODEL), jnp.float32),
        in_specs=[pl.BlockSpec(memory_space=pltpu.VMEM)] * 5,
        out_specs=pl.BlockSpec(memory_space=pltpu.VMEM),
        scratch_shapes=[
            pltpu.VMEM((N_DEV, D_MODEL, H_CHUNK), BF),
            pltpu.VMEM((N_DEV, H_CHUNK, D_MODEL), BF),
            pltpu.VMEM((SQ, H_CHUNK), BF),
            pltpu.SemaphoreType.DMA((2, 2)),
            pltpu.SemaphoreType.DMA((2, 2)),
            pltpu.SemaphoreType.DMA((2, 2)),
            pltpu.SemaphoreType.DMA((2, 2)),
        ],
        compiler_params=pltpu.CompilerParams(collective_id=0),
    )(xb, wq_b, Kt, Vt, wo_b)


# device time: 23877 ns/iter; 2.4956x vs baseline; 1.1362x over previous
import jax
import jax.numpy as jnp
from jax import lax
from jax.experimental import pallas as pl
from jax.experimental.pallas import tpu as pltpu

N_DEV = 4
B_LOC = 2
SQ = 256
SKV = 256
HQ = 16
H_LOC = HQ // N_DEV
DH = 64
D_MODEL = 512
D_QK = HQ * DH
H_CHUNK = D_QK // N_DEV
BF = jnp.bfloat16


def kernel(x, Wq, K_ext, V_ext, Wo):
    pos = lax.axis_index("i")
    K_loc = lax.dynamic_slice_in_dim(K_ext, pos * B_LOC, B_LOC, axis=0)
    V_loc = lax.dynamic_slice_in_dim(V_ext, pos * B_LOC, B_LOC, axis=0)

    def arrange(a):
        a = a.reshape(B_LOC, SKV, N_DEV, H_LOC, DH)
        return a.transpose(2, 0, 3, 1, 4).astype(BF)

    Kt = arrange(K_loc)
    Vt = arrange(V_loc)

    def body(x_ref, wq_ref, k_ref, v_ref, wo_ref, out_ref,
             wq_g, wo_g, x_scr, ctx_scr, wq_ss, wq_rs, wo_ss, wo_rs):
        my = lax.axis_index("i")
        left = lax.rem(my + (N_DEV - 1), N_DEV)
        right = lax.rem(my + 1, N_DEV)
        opp = lax.rem(my + 2, N_DEV)

        barrier = pltpu.get_barrier_semaphore()
        for nbr in (left, right):
            pl.semaphore_signal(
                barrier, inc=1,
                device_id=(nbr,), device_id_type=pl.DeviceIdType.MESH,
            )
        pl.semaphore_wait(barrier, 2)

        row = lax.broadcasted_iota(jnp.int32, (SQ, SKV), 0) // 64
        col = lax.broadcasted_iota(jnp.int32, (SQ, SKV), 1) // 64
        mask = (row == col) | (col == 0) | (lax.rem(row + col, 3) == 0)
        neg = jnp.float32(-1e9)

        def mm(a, b):
            return lax.dot_general(
                a, b, dimension_numbers=(((1,), (0,)), ((), ())),
                preferred_element_type=jnp.float32,
            )

        def mm_t(a, b):
            return lax.dot_general(
                a, b, dimension_numbers=(((1,), (1,)), ((), ())),
                preferred_element_type=jnp.float32,
            )

        def do_chunk(p, wq_c, wo_c, first):
            for b in range(B_LOC):
                qp = mm(x_scr[b], wq_c)
                for hh in range(H_LOC):
                    q = qp[:, hh * DH:(hh + 1) * DH].astype(BF)
                    k = k_ref[p, b, hh]
                    s = mm_t(q, k) * jnp.float32(0.125)
                    s = jnp.where(mask, s, neg)
                    m = jnp.max(s, axis=1, keepdims=True)
                    w = jnp.exp(s - m)
                    w = (w / jnp.sum(w, axis=1, keepdims=True)).astype(BF)
                    ctx_scr[:, hh * DH:(hh + 1) * DH] = mm(
                        w, v_ref[p, b, hh]).astype(BF)
                partial = mm(ctx_scr[...], wo_c)
                if first:
                    out_ref[b, :, :] = partial
                else:
                    out_ref[b, :, :] = out_ref[b, :, :] + partial

        wq_g[my] = wq_ref[...].astype(BF)
        wo_g[my] = wo_ref[...].astype(BF)
        for b in range(B_LOC):
            x_scr[b] = x_ref[b].astype(BF)

        hop0 = []
        for d, tgt in ((0, right), (1, left)):
            rq = pltpu.make_async_remote_copy(
                src_ref=wq_g.at[my], dst_ref=wq_g.at[my],
                send_sem=wq_ss.at[0, d], recv_sem=wq_rs.at[0, d],
                device_id=(tgt,), device_id_type=pl.DeviceIdType.MESH,
            )
            ro = pltpu.make_async_remote_copy(
                src_ref=wo_g.at[my], dst_ref=wo_g.at[my],
                send_sem=wo_ss.at[0, d], recv_sem=wo_rs.at[0, d],
                device_id=(tgt,), device_id_type=pl.DeviceIdType.MESH,
            )
            rq.start()
            ro.start()
            hop0 += [rq, ro]

        do_chunk(my, wq_g[my], wo_g[my], first=True)

        for r in hop0:
            r.wait()

        hop1 = []
        rq = pltpu.make_async_remote_copy(
            src_ref=wq_g.at[left, pl.ds(0, D_MODEL // 2)],
            dst_ref=wq_g.at[left, pl.ds(0, D_MODEL // 2)],
            send_sem=wq_ss.at[1, 0], recv_sem=wq_rs.at[1, 0],
            device_id=(right,), device_id_type=pl.DeviceIdType.MESH,
        )
        ro = pltpu.make_async_remote_copy(
            src_ref=wo_g.at[left, pl.ds(0, H_CHUNK // 2)],
            dst_ref=wo_g.at[left, pl.ds(0, H_CHUNK // 2)],
            send_sem=wo_ss.at[1, 0], recv_sem=wo_rs.at[1, 0],
            device_id=(right,), device_id_type=pl.DeviceIdType.MESH,
        )
        rq.start()
        ro.start()
        hop1 += [rq, ro]
        rq = pltpu.make_async_remote_copy(
            src_ref=wq_g.at[right, pl.ds(D_MODEL // 2, D_MODEL // 2)],
            dst_ref=wq_g.at[right, pl.ds(D_MODEL // 2, D_MODEL // 2)],
            send_sem=wq_ss.at[1, 1], recv_sem=wq_rs.at[1, 1],
            device_id=(left,), device_id_type=pl.DeviceIdType.MESH,
        )
        ro = pltpu.make_async_remote_copy(
            src_ref=wo_g.at[right, pl.ds(H_CHUNK // 2, H_CHUNK // 2)],
            dst_ref=wo_g.at[right, pl.ds(H_CHUNK // 2, H_CHUNK // 2)],
            send_sem=wo_ss.at[1, 1], recv_sem=wo_rs.at[1, 1],
            device_id=(left,), device_id_type=pl.DeviceIdType.MESH,
        )
        rq.start()
        ro.start()
        hop1 += [rq, ro]

        do_chunk(left, wq_g[left], wo_g[left], first=False)
        do_chunk(right, wq_g[right], wo_g[right], first=False)

        for r in hop1:
            r.wait()

        do_chunk(opp, wq_g[opp], wo_g[opp], first=False)

    return pl.pallas_call(
        body,
        out_shape=jax.ShapeDtypeStruct((B_LOC, SQ, D_MODEL), jnp.float32),
        in_specs=[pl.BlockSpec(memory_space=pltpu.VMEM)] * 5,
        out_specs=pl.BlockSpec(memory_space=pltpu.VMEM),
        scratch_shapes=[
            pltpu.VMEM((N_DEV, D_MODEL, H_CHUNK), BF),
            pltpu.VMEM((N_DEV, H_CHUNK, D_MODEL), BF),
            pltpu.VMEM((B_LOC, SQ, D_MODEL), BF),
            pltpu.VMEM((SQ, H_CHUNK), BF),
            pltpu.SemaphoreType.DMA((2, 2)),
            pltpu.SemaphoreType.DMA((2, 2)),
            pltpu.SemaphoreType.DMA((2, 2)),
            pltpu.SemaphoreType.DMA((2, 2)),
        ],
        compiler_params=pltpu.CompilerParams(collective_id=0),
    )(x, Wq, Kt, Vt, Wo)


# device time: 22872 ns/iter; 2.6052x vs baseline; 1.0439x over previous
import jax
import jax.numpy as jnp
from jax import lax
from jax.experimental import pallas as pl
from jax.experimental.pallas import tpu as pltpu

N_DEV = 4
B_LOC = 2
SQ = 256
SKV = 256
HQ = 16
H_LOC = HQ // N_DEV
DH = 64
D_MODEL = 512
D_QK = HQ * DH
H_CHUNK = D_QK // N_DEV
BF = jnp.bfloat16


def kernel(x, Wq, K_ext, V_ext, Wo):
    pos = lax.axis_index("i")
    K_loc = lax.dynamic_slice_in_dim(K_ext, pos * B_LOC, B_LOC, axis=0)
    V_loc = lax.dynamic_slice_in_dim(V_ext, pos * B_LOC, B_LOC, axis=0)

    def arrange(a):
        a = a.reshape(B_LOC, SKV, N_DEV, H_LOC, DH)
        return a.transpose(2, 0, 3, 1, 4).astype(BF)

    Kt = arrange(K_loc)
    Vt = arrange(V_loc)

    def body(x_ref, wq_ref, k_ref, v_ref, wo_ref, out_ref,
             wq_g, wo_g, x_scr, ctx_scr, wq_ss, wq_rs, wo_ss, wo_rs):
        my = lax.axis_index("i")
        left = lax.rem(my + (N_DEV - 1), N_DEV)
        right = lax.rem(my + 1, N_DEV)
        opp = lax.rem(my + 2, N_DEV)

        barrier = pltpu.get_barrier_semaphore()
        for nbr in (left, right):
            pl.semaphore_signal(
                barrier, inc=1,
                device_id=(nbr,), device_id_type=pl.DeviceIdType.MESH,
            )
        pl.semaphore_wait(barrier, 2)

        row = lax.broadcasted_iota(jnp.int32, (SQ, SKV), 0) // 64
        col = lax.broadcasted_iota(jnp.int32, (SQ, SKV), 1) // 64
        mask = (row == col) | (col == 0) | (lax.rem(row + col, 3) == 0)
        neg = jnp.float32(-1e9)

        def mm(a, b):
            return lax.dot_general(
                a, b, dimension_numbers=(((1,), (0,)), ((), ())),
                preferred_element_type=jnp.float32,
            )

        def mm_t(a, b):
            return lax.dot_general(
                a, b, dimension_numbers=(((1,), (1,)), ((), ())),
                preferred_element_type=jnp.float32,
            )

        def do_chunk(p, wq_c, wo_c, first):
            for b in range(B_LOC):
                qp = mm(x_scr[b], wq_c)
                for hh in range(H_LOC):
                    q = (qp[:, hh * DH:(hh + 1) * DH]
                         * jnp.float32(0.125)).astype(BF)
                    k = k_ref[p, b, hh]
                    s = mm_t(q, k)
                    w = jnp.exp(jnp.where(mask, s, neg))
                    inv = 1.0 / jnp.sum(w, axis=1, keepdims=True)
                    w = (w * inv).astype(BF)
                    ctx_scr[:, hh * DH:(hh + 1) * DH] = mm(
                        w, v_ref[p, b, hh]).astype(BF)
                partial = mm(ctx_scr[...], wo_c)
                if first:
                    out_ref[b, :, :] = partial
                else:
                    out_ref[b, :, :] = out_ref[b, :, :] + partial

        wq_g[my] = wq_ref[...].astype(BF)
        wo_g[my] = wo_ref[...].astype(BF)

        hop0 = []
        for d, tgt in ((0, right), (1, left)):
            rq = pltpu.make_async_remote_copy(
                src_ref=wq_g.at[my], dst_ref=wq_g.at[my],
                send_sem=wq_ss.at[0, d], recv_sem=wq_rs.at[0, d],
                device_id=(tgt,), device_id_type=pl.DeviceIdType.MESH,
            )
            ro = pltpu.make_async_remote_copy(
                src_ref=wo_g.at[my], dst_ref=wo_g.at[my],
                send_sem=wo_ss.at[0, d], recv_sem=wo_rs.at[0, d],
                device_id=(tgt,), device_id_type=pl.DeviceIdType.MESH,
            )
            rq.start()
            ro.start()
            hop0 += [rq, ro]

        for b in range(B_LOC):
            x_scr[b] = x_ref[b].astype(BF)

        do_chunk(my, wq_g[my], wo_g[my], first=True)

        for r in hop0:
            r.wait()

        hop1 = []
        rq = pltpu.make_async_remote_copy(
            src_ref=wq_g.at[left, pl.ds(0, D_MODEL // 2)],
            dst_ref=wq_g.at[left, pl.ds(0, D_MODEL // 2)],
            send_sem=wq_ss.at[1, 0], recv_sem=wq_rs.at[1, 0],
            device_id=(right,), device_id_type=pl.DeviceIdType.MESH,
        )
        ro = pltpu.make_async_remote_copy(
            src_ref=wo_g.at[left, pl.ds(0, H_CHUNK // 2)],
            dst_ref=wo_g.at[left, pl.ds(0, H_CHUNK // 2)],
            send_sem=wo_ss.at[1, 0], recv_sem=wo_rs.at[1, 0],
            device_id=(right,), device_id_type=pl.DeviceIdType.MESH,
        )
        rq.start()
        ro.start()
        hop1 += [rq, ro]
        rq = pltpu.make_async_remote_copy(
            src_ref=wq_g.at[right, pl.ds(D_MODEL // 2, D_MODEL // 2)],
            dst_ref=wq_g.at[right, pl.ds(D_MODEL // 2, D_MODEL // 2)],
            send_sem=wq_ss.at[1, 1], recv_sem=wq_rs.at[1, 1],
            device_id=(left,), device_id_type=pl.DeviceIdType.MESH,
        )
        ro = pltpu.make_async_remote_copy(
            src_ref=wo_g.at[right, pl.ds(H_CHUNK // 2, H_CHUNK // 2)],
            dst_ref=wo_g.at[right, pl.ds(H_CHUNK // 2, H_CHUNK // 2)],
            send_sem=wo_ss.at[1, 1], recv_sem=wo_rs.at[1, 1],
            device_id=(left,), device_id_type=pl.DeviceIdType.MESH,
        )
        rq.start()
        ro.start()
        hop1 += [rq, ro]

        do_chunk(left, wq_g[left], wo_g[left], first=False)
        do_chunk(right, wq_g[right], wo_g[right], first=False)

        for r in hop1:
            r.wait()

        do_chunk(opp, wq_g[opp], wo_g[opp], first=False)

    return pl.pallas_call(
        body,
        out_shape=jax.ShapeDtypeStruct((B_LOC, SQ, D_MODEL), jnp.float32),
        in_specs=[pl.BlockSpec(memory_space=pltpu.VMEM)] * 5,
        out_specs=pl.BlockSpec(memory_space=pltpu.VMEM),
        scratch_shapes=[
            pltpu.VMEM((N_DEV, D_MODEL, H_CHUNK), BF),
            pltpu.VMEM((N_DEV, H_CHUNK, D_MODEL), BF),
            pltpu.VMEM((B_LOC, SQ, D_MODEL), BF),
            pltpu.VMEM((SQ, H_CHUNK), BF),
            pltpu.SemaphoreType.DMA((2, 2)),
            pltpu.SemaphoreType.DMA((2, 2)),
            pltpu.SemaphoreType.DMA((2, 2)),
            pltpu.SemaphoreType.DMA((2, 2)),
        ],
        compiler_params=pltpu.CompilerParams(collective_id=0),
    )(x, Wq, Kt, Vt, Wo)
